# Initial kernel scaffold; baseline (speedup 1.0000x reference)
import jax
import jax.numpy as jnp
from jax.experimental import pallas as pl


def kernel(coords, feats):
    raise NotImplementedError("write your pallas kernel here")


# speedup vs baseline: 1.4957x; 1.4957x over previous
import jax
import jax.numpy as jnp
from jax import lax
from jax.experimental import pallas as pl
from jax.experimental.pallas import tpu as pltpu
from jax.experimental.pallas import tpu_sc as plsc

_STRIDE_SHIFT = 3
_BATCH = 2
_NX, _NH, _NZ = 200, 4, 176
_ROWS_PER_BATCH = _NX * _NZ * _NH
_TOTAL_ROWS = _BATCH * _ROWS_PER_BATCH
_C = 128

_NCORES = 2
_NTILES = 16
_L = 16
_NPASS = 11
_R = _TOTAL_ROWS // (_NPASS * _NCORES)
_ACC_ROWS = _R + _L
_ZROWS = _ACC_ROWS // _NTILES
_FROWS = _R // _NTILES

_NPAD = 100352
_PTS_PER_TILE = _NPAD // _NTILES
_VREGS_PER_TILE = _PTS_PER_TILE // _L
_BIDX = 64
_LIST_CAP = 6400


def _sc_scatter_body(coords_t, feats, out, acc, r_v, loc_flat, pid_flat,
                     pid_row, loc_row, rows_v, sem):
    core = lax.axis_index("c")
    tile = lax.axis_index("s")
    pbase = tile * _PTS_PER_TILE
    lane = lax.iota(jnp.int32, _L)

    stage = pid_flat

    def _accum_component(row, fn, first):
        pltpu.sync_copy(coords_t.at[row, pl.ds(pbase, _PTS_PER_TILE)],
                        stage.at[pl.ds(0, _PTS_PER_TILE)])

        def step(i, _):
            off = i * _L
            v = fn(stage[pl.ds(off, _L)])
            r_v[pl.ds(off, _L)] = v if first else r_v[pl.ds(off, _L)] + v
            return 0
        lax.fori_loop(0, _VREGS_PER_TILE, step, 0)

    _accum_component(
        3, lambda cb: cb * _ROWS_PER_BATCH, True)
    _accum_component(
        0, lambda cx: lax.shift_right_logical(cx, _STRIDE_SHIFT) * (_NZ * _NH),
        False)
    _accum_component(
        2, lambda cz: lax.shift_right_logical(cz, _STRIDE_SHIFT) * _NH, False)
    _accum_component(
        1, lambda ch: jnp.clip(lax.shift_right_logical(ch, _STRIDE_SHIFT),
                               0, _NH - 1), False)

    def one_pass(p, _):
        base = (p * _NCORES + core) * _R

        zero16 = jnp.zeros((_L,), jnp.float32)

        def zrow(i, _):
            for c in range(_C // _L):
                rows_v[i, pl.ds(c * _L, _L)] = zero16
            return 0
        lax.fori_loop(0, _BIDX, zrow, 0)

        zbase = tile * _ZROWS
        for k in range(_ZROWS // _BIDX):
            pltpu.sync_copy(rows_v, acc.at[pl.ds(zbase + k * _BIDX, _BIDX)])
        rem = _ZROWS % _BIDX
        if rem:
            pltpu.sync_copy(rows_v.at[pl.ds(0, rem)],
                            acc.at[pl.ds(zbase + _ZROWS - rem, rem)])

        def compact(i, ptr):
            off = i * _L
            r = r_v[pl.ds(off, _L)]
            loc = r - base
            mask = (loc >= 0) & (loc < _R)
            mi = mask.astype(jnp.int32)
            cum = plsc.cumsum(mi)
            pos = ptr + cum - 1
            pid = pbase + off + lane
            plsc.store_scatter(loc_flat, [pos], loc, mask=mask)
            plsc.store_scatter(pid_flat, [pos], pid, mask=mask)
            return ptr + jnp.sum(mi)
        m = lax.fori_loop(0, _VREGS_PER_TILE, compact, jnp.int32(0))

        for k in range(_BIDX // _L + 1):
            loc_flat[pl.ds(m + k * _L, _L)] = _R + lane
            pid_flat[pl.ds(m + k * _L, _L)] = lane

        plsc.subcore_barrier()

        nb = (m + _BIDX - 1) // _BIDX

        def one_batch(j, _):
            fbase = j * _BIDX
            for b in range(_BIDX // _L):
                pid_row[pl.ds(b * _L, _L)] = pid_flat[pl.ds(fbase + b * _L, _L)]
                loc_row[pl.ds(b * _L, _L)] = loc_flat[pl.ds(fbase + b * _L, _L)]
            pltpu.async_copy(feats.at[pid_row], rows_v, sem).wait()
            pltpu.sync_copy(rows_v, acc.at[loc_row], add=True)
            return 0
        lax.fori_loop(0, nb, one_batch, 0)

        plsc.subcore_barrier()

        fbase = tile * _FROWS
        pltpu.sync_copy(acc.at[pl.ds(fbase, _FROWS)],
                        out.at[pl.ds(base + fbase, _FROWS)])

        plsc.subcore_barrier()
        return 0

    lax.fori_loop(0, _NPASS, one_pass, 0)


@jax.jit
def _sc_scatter(coords_t, feats):
    mesh = plsc.VectorSubcoreMesh(core_axis_name="c", subcore_axis_name="s")
    fn = pl.kernel(
        _sc_scatter_body,
        out_type=jax.ShapeDtypeStruct((_TOTAL_ROWS, _C), jnp.float32),
        mesh=mesh,
        compiler_params=pltpu.CompilerParams(needs_layout_passes=False),
        scratch_types=[
            pltpu.VMEM_SHARED((_ACC_ROWS, _C), jnp.float32),
            pltpu.VMEM((_PTS_PER_TILE,), jnp.int32),
            pltpu.VMEM((_LIST_CAP,), jnp.int32),
            pltpu.VMEM((_LIST_CAP,), jnp.int32),
            pltpu.VMEM((_BIDX,), jnp.int32),
            pltpu.VMEM((_BIDX,), jnp.int32),
            pltpu.VMEM((_BIDX, _C), jnp.float32),
            pltpu.SemaphoreType.DMA,
        ],
    )
    return fn(coords_t, feats)


def kernel(coords, feats):
    n = coords.shape[0]
    pad = jnp.zeros((_NPAD - n, 4), jnp.int32).at[:, 3].set(_BATCH)
    coords_t = jnp.concatenate([coords.astype(jnp.int32), pad], axis=0).T
    table = _sc_scatter(coords_t, feats)
    out = table.reshape(_BATCH, _NX, _NZ, _NH * _C)
    return jnp.transpose(out, (0, 3, 1, 2))
